# baseline (speedup 1.0000x reference)
import functools

import jax
import jax.numpy as jnp
from jax import lax
from jax.experimental import pallas as pl
from jax.experimental.pallas import tpu as pltpu
from jax.experimental.pallas import tpu_sc as plsc

B = 16384
EMB = 256
SRC = 256
OUT = 2 * SRC

NC = 2
NS = 16
NW = NC * NS

NCHUNK = 4
BC = B // NCHUNK
B_PER_W = BC // NW

BM = 2048


def _sc_gather_body(table_hbm, idx_hbm, out_hbm, idx_v, rows_v, sem):
    wid = lax.axis_index("s") * NC + lax.axis_index("c")
    base = wid * B_PER_W
    pltpu.sync_copy(idx_hbm.at[pl.ds(base, B_PER_W)], idx_v)
    pltpu.async_copy(table_hbm.at[idx_v], rows_v, sem).wait()
    pltpu.sync_copy(rows_v, out_hbm.at[pl.ds(base, B_PER_W)])


_sc_gather = pl.kernel(
    _sc_gather_body,
    out_type=jax.ShapeDtypeStruct((BC, EMB), jnp.float32),
    mesh=plsc.VectorSubcoreMesh(core_axis_name="c", subcore_axis_name="s"),
    scratch_types=[
        pltpu.VMEM((B_PER_W,), jnp.int32),
        pltpu.VMEM((B_PER_W, EMB), jnp.float32),
        pltpu.SemaphoreType.DMA,
    ],
)


def _mm_body(x_ref, w_ref, o_ref):
    o_ref[...] = jnp.dot(x_ref[...], w_ref[...],
                         preferred_element_type=jnp.float32)


_matmul = pl.pallas_call(
    _mm_body,
    grid=(BC // BM,),
    in_specs=[
        pl.BlockSpec((BM, EMB), lambda i: (i, 0)),
        pl.BlockSpec((EMB, OUT), lambda i: (0, 0)),
    ],
    out_specs=pl.BlockSpec((BM, OUT), lambda i: (i, 0)),
    out_shape=jax.ShapeDtypeStruct((BC, OUT), jnp.float32),
)


@jax.jit
def _run(indexes, entity_table, wc):
    outs = []
    for c in range(NCHUNK):
        emb_c = _sc_gather(entity_table, lax.slice(indexes, (c * BC,), ((c + 1) * BC,)))
        outs.append(_matmul(emb_c, wc))
    return jnp.concatenate(outs, axis=0).reshape(B, 2, SRC)


def kernel(indexes, entity_table, W0, W1):
    wc = jnp.concatenate([W0, W1], axis=0).T
    return _run(indexes, entity_table, wc)


# speedup vs baseline: 1.3541x; 1.3541x over previous
import functools

import jax
import jax.numpy as jnp
from jax import lax
from jax.experimental import pallas as pl
from jax.experimental.pallas import tpu as pltpu
from jax.experimental.pallas import tpu_sc as plsc

B = 16384
EMB = 256
SRC = 256
OUT = 2 * SRC

NC = 2
NS = 16
NW = NC * NS
B_PER_W = B // NW
CH = 128
NCH = B_PER_W // CH

BM = 2048


def _sc_gather_body(table_hbm, idx_hbm, out_hbm, idx_v, buf0, buf1,
                    sem_g, sem_s0, sem_s1):
    wid = lax.axis_index("s") * NC + lax.axis_index("c")
    base = wid * B_PER_W
    pltpu.sync_copy(idx_hbm.at[pl.ds(base, B_PER_W)], idx_v)
    bufs = (buf0, buf1)
    sems = (sem_s0, sem_s1)
    scat = [None, None]
    g = pltpu.async_copy(table_hbm.at[idx_v.at[pl.ds(0, CH)]], buf0, sem_g)
    for c in range(NCH):
        g.wait()
        if c + 1 < NCH:
            nxt = (c + 1) % 2
            if scat[nxt] is not None:
                scat[nxt].wait()
            g = pltpu.async_copy(
                table_hbm.at[idx_v.at[pl.ds((c + 1) * CH, CH)]],
                bufs[nxt], sem_g)
        scat[c % 2] = pltpu.async_copy(
            bufs[c % 2], out_hbm.at[pl.ds(base + c * CH, CH)], sems[c % 2])
    scat[0].wait()
    scat[1].wait()


_sc_gather = pl.kernel(
    _sc_gather_body,
    out_type=jax.ShapeDtypeStruct((B, EMB), jnp.float32),
    mesh=plsc.VectorSubcoreMesh(core_axis_name="c", subcore_axis_name="s"),
    scratch_types=[
        pltpu.VMEM((B_PER_W,), jnp.int32),
        pltpu.VMEM((CH, EMB), jnp.float32),
        pltpu.VMEM((CH, EMB), jnp.float32),
        pltpu.SemaphoreType.DMA,
        pltpu.SemaphoreType.DMA,
        pltpu.SemaphoreType.DMA,
    ],
)


def _mm_body(x_ref, w_ref, o_ref):
    o_ref[...] = jnp.dot(x_ref[...], w_ref[...],
                         preferred_element_type=jnp.float32)


_matmul = pl.pallas_call(
    _mm_body,
    grid=(B // BM,),
    in_specs=[
        pl.BlockSpec((BM, EMB), lambda i: (i, 0)),
        pl.BlockSpec((EMB, OUT), lambda i: (0, 0)),
    ],
    out_specs=pl.BlockSpec((BM, OUT), lambda i: (i, 0)),
    out_shape=jax.ShapeDtypeStruct((B, OUT), jnp.float32),
)


@jax.jit
def _run(indexes, entity_table, wc):
    emb = _sc_gather(entity_table, indexes)
    return _matmul(emb, wc).reshape(B, 2, SRC)


def kernel(indexes, entity_table, W0, W1):
    wc = jnp.concatenate([W0, W1], axis=0).T
    return _run(indexes, entity_table, wc)


# speedup vs baseline: 1.3673x; 1.0098x over previous
import functools

import jax
import jax.numpy as jnp
from jax import lax
from jax.experimental import pallas as pl
from jax.experimental.pallas import tpu as pltpu
from jax.experimental.pallas import tpu_sc as plsc

B = 16384
EMB = 256
SRC = 256
OUT = 2 * SRC

NC = 2
NS = 16
NW = NC * NS
B_PER_W = B // NW
CH = 128
NCH = B_PER_W // CH

BM = 4096


def _sc_gather_body(table_hbm, idx_hbm, out_hbm, idx_v, buf0, buf1,
                    sem_g, sem_s0, sem_s1):
    wid = lax.axis_index("s") * NC + lax.axis_index("c")
    base = wid * B_PER_W
    pltpu.sync_copy(idx_hbm.at[pl.ds(base, B_PER_W)], idx_v)
    bufs = (buf0, buf1)
    sems = (sem_s0, sem_s1)
    scat = [None, None]
    g = pltpu.async_copy(table_hbm.at[idx_v.at[pl.ds(0, CH)]], buf0, sem_g)
    for c in range(NCH):
        g.wait()
        if c + 1 < NCH:
            nxt = (c + 1) % 2
            if scat[nxt] is not None:
                scat[nxt].wait()
            g = pltpu.async_copy(
                table_hbm.at[idx_v.at[pl.ds((c + 1) * CH, CH)]],
                bufs[nxt], sem_g)
        scat[c % 2] = pltpu.async_copy(
            bufs[c % 2], out_hbm.at[pl.ds(base + c * CH, CH)], sems[c % 2])
    scat[0].wait()
    scat[1].wait()


_sc_gather = pl.kernel(
    _sc_gather_body,
    out_type=jax.ShapeDtypeStruct((B, EMB), jnp.float32),
    mesh=plsc.VectorSubcoreMesh(core_axis_name="c", subcore_axis_name="s"),
    scratch_types=[
        pltpu.VMEM((B_PER_W,), jnp.int32),
        pltpu.VMEM((CH, EMB), jnp.float32),
        pltpu.VMEM((CH, EMB), jnp.float32),
        pltpu.SemaphoreType.DMA,
        pltpu.SemaphoreType.DMA,
        pltpu.SemaphoreType.DMA,
    ],
)


def _mm_body(x_ref, w_ref, o_ref):
    o_ref[...] = jnp.dot(x_ref[...], w_ref[...],
                         preferred_element_type=jnp.float32)


_matmul = pl.pallas_call(
    _mm_body,
    grid=(B // BM,),
    in_specs=[
        pl.BlockSpec((BM, EMB), lambda i: (i, 0)),
        pl.BlockSpec((EMB, OUT), lambda i: (0, 0)),
    ],
    out_specs=pl.BlockSpec((BM, OUT), lambda i: (i, 0)),
    out_shape=jax.ShapeDtypeStruct((B, OUT), jnp.float32),
)


@jax.jit
def _run(indexes, entity_table, wc):
    emb = _sc_gather(entity_table, indexes)
    return _matmul(emb, wc).reshape(B, 2, SRC)


def kernel(indexes, entity_table, W0, W1):
    wc = jnp.concatenate([W0, W1], axis=0).T
    return _run(indexes, entity_table, wc)
